# baseline (speedup 1.0000x reference)
import functools

import numpy as np
import jax
import jax.numpy as jnp
from jax import lax
from jax.experimental import pallas as pl
from jax.experimental.pallas import tpu as pltpu
from jax.experimental.pallas import tpu_sc as plsc

N = 10000
K = 16
D = 128
B = 4
OUT = 2

NC, NS, L = 2, 16, 16
NW = NC * NS
ROWW = B * D
NPAD = 10240
PER_W = NPAD // NW
CHUNK = 8
N_CHUNKS = PER_W // CHUNK
NROWS = NPAD * B
CK = CHUNK * K
NH = N_CHUNKS // 2


def _stored_perm(n):
    idx = np.zeros(n, np.int32)
    for t in range(n // 32):
        for i in range(16):
            idx[32 * t + i] = 32 * t + 2 * i
            idx[32 * t + 16 + i] = 32 * t + 2 * i + 1
    return idx


_NAT128 = _stored_perm(D)
_NAT512 = _stored_perm(ROWW)


def _gather_agg_body(x_hbm, adj_hbm, w_hbm, out_hbm, adj_v, rows_a, rows_b,
                     acc_a, acc_b, w_v, sem_a, sem_b, sem_oa, sem_ob):
    cid = lax.axis_index("c")
    sid = lax.axis_index("s")
    wid = sid * NC + cid
    base = wid * PER_W
    pltpu.sync_copy(w_hbm, w_v)
    pltpu.sync_copy(adj_hbm.at[wid], adj_v)

    def gather(c, buf, sem):
        return pltpu.async_copy(x_hbm.at[adj_v.at[c]], buf, sem)

    def wait_gather(buf, sem):
        pltpu.make_async_copy(x_hbm.at[pl.ds(0, CK)], buf, sem).wait()

    def wait_scatter(acc, sem):
        pltpu.make_async_copy(acc, out_hbm.at[pl.ds(0, CHUNK)], sem).wait()

    def compute(c, buf, acc):
        def vbody(v, carry):
            sle = pl.ds(v * 32, L)
            slo = pl.ds(v * 32 + L, L)
            we = [w_v[k, sle] for k in range(K)]
            wo = [w_v[k, slo] for k in range(K)]
            for j in range(CHUNK):
                r0 = j * K
                xi = buf[r0, pl.ds(v * L, L)]
                te = plsc.bitcast(xi << 16, jnp.float32) * we[0]
                to = plsc.bitcast(xi & -65536, jnp.float32) * wo[0]
                for k in range(1, K):
                    xi = buf[r0 + k, pl.ds(v * L, L)]
                    te = te + plsc.bitcast(xi << 16, jnp.float32) * we[k]
                    to = to + plsc.bitcast(xi & -65536, jnp.float32) * wo[k]
                acc[j, sle] = te
                acc[j, slo] = to
            return carry

        lax.fori_loop(0, ROWW // 32, vbody, 0)
        return pltpu.async_copy(
            acc, out_hbm.at[pl.ds(base + c * CHUNK, CHUNK)],
            sem_oa if acc is acc_a else sem_ob)

    gather(0, rows_a, sem_a)

    def body(t, carry):
        c0 = 2 * t
        c1 = 2 * t + 1
        gather(c1, rows_b, sem_b)
        wait_gather(rows_a, sem_a)

        @pl.when(t > 0)
        def _():
            wait_scatter(acc_a, sem_oa)
        compute(c0, rows_a, acc_a)

        @pl.when(t < NH - 1)
        def _():
            gather(c0 + 2, rows_a, sem_a)
        wait_gather(rows_b, sem_b)

        @pl.when(t > 0)
        def _():
            wait_scatter(acc_b, sem_ob)
        compute(c1, rows_b, acc_b)
        return carry

    lax.fori_loop(0, NH, body, 0)
    wait_scatter(acc_a, sem_oa)
    wait_scatter(acc_b, sem_ob)


@functools.partial(jax.jit, static_argnames=())
def _gather_agg(xb_rows, adj_w, w_stored):
    mesh = plsc.VectorSubcoreMesh(core_axis_name="c", subcore_axis_name="s")
    return pl.kernel(
        _gather_agg_body,
        out_type=jax.ShapeDtypeStruct((NPAD, ROWW), jnp.float32),
        mesh=mesh,
        compiler_params=pltpu.CompilerParams(needs_layout_passes=False),
        scratch_types=[
            pltpu.VMEM((N_CHUNKS, CK), jnp.int32),
            pltpu.VMEM((CK, ROWW // 2), jnp.int32),
            pltpu.VMEM((CK, ROWW // 2), jnp.int32),
            pltpu.VMEM((CHUNK, ROWW), jnp.float32),
            pltpu.VMEM((CHUNK, ROWW), jnp.float32),
            pltpu.VMEM((K, ROWW), jnp.float32),
            pltpu.SemaphoreType.DMA,
            pltpu.SemaphoreType.DMA,
            pltpu.SemaphoreType.DMA,
            pltpu.SemaphoreType.DMA,
        ],
    )(xb_rows, adj_w, w_stored)


def _init_body(lat_ref, win_ref, bin_ref, pos_ref, out_ref, outb_ref):
    x0 = jnp.dot(lat_ref[...], win_ref[...],
                 preferred_element_type=jnp.float32) + bin_ref[...]
    x = pos_ref[...][:, None, :] + x0[None, :, :]
    out_ref[...] = x
    outb_ref[...] = x.astype(jnp.bfloat16)


def _init_x(latent, W_in, b_in, pos_pad):
    tn = 1024
    return pl.pallas_call(
        _init_body,
        grid=(NPAD // tn,),
        in_specs=[
            pl.BlockSpec((B, W_in.shape[0]), lambda i: (0, 0)),
            pl.BlockSpec((W_in.shape[0], D), lambda i: (0, 0)),
            pl.BlockSpec((1, D), lambda i: (0, 0)),
            pl.BlockSpec((tn, D), lambda i: (i, 0)),
        ],
        out_specs=[
            pl.BlockSpec((tn, B, D), lambda i: (i, 0, 0)),
            pl.BlockSpec((tn, B, D), lambda i: (i, 0, 0)),
        ],
        out_shape=[
            jax.ShapeDtypeStruct((NPAD, B, D), jnp.float32),
            jax.ShapeDtypeStruct((NPAD, B, D), jnp.bfloat16),
        ],
    )(latent, W_in, b_in.reshape(1, D), pos_pad)


def _ln(x, g, b):
    m = jnp.mean(x, axis=-1, keepdims=True)
    v = jnp.mean((x - m) ** 2, axis=-1, keepdims=True)
    return (x - m) * lax.rsqrt(v + 1e-5) * g + b


def _block_body(x_ref, agg_ref, g_ref, b_ref, w1_ref, b1_ref, w2_ref, b2_ref,
                out_ref, outb_ref):
    h = _ln(agg_ref[...], g_ref[...], b_ref[...])
    u = jax.nn.gelu(jnp.dot(h, w1_ref[...], preferred_element_type=jnp.float32)
                    + b1_ref[...])
    y = jnp.dot(u, w2_ref[...], preferred_element_type=jnp.float32) + b2_ref[...]
    x = x_ref[...] + y
    out_ref[...] = x
    outb_ref[...] = x.astype(jnp.bfloat16)


def _mlp_block(x2d, agg2d, g, b, W1, b1, W2, b2):
    r = 2048
    h4 = 4 * D
    return pl.pallas_call(
        _block_body,
        grid=(NROWS // r,),
        in_specs=[
            pl.BlockSpec((r, D), lambda i: (i, 0)),
            pl.BlockSpec((r, D), lambda i: (i, 0)),
            pl.BlockSpec((1, D), lambda i: (0, 0)),
            pl.BlockSpec((1, D), lambda i: (0, 0)),
            pl.BlockSpec((D, h4), lambda i: (0, 0)),
            pl.BlockSpec((1, h4), lambda i: (0, 0)),
            pl.BlockSpec((h4, D), lambda i: (0, 0)),
            pl.BlockSpec((1, D), lambda i: (0, 0)),
        ],
        out_specs=[
            pl.BlockSpec((r, D), lambda i: (i, 0)),
            pl.BlockSpec((r, D), lambda i: (i, 0)),
        ],
        out_shape=[
            jax.ShapeDtypeStruct((NROWS, D), jnp.float32),
            jax.ShapeDtypeStruct((NROWS, D), jnp.bfloat16),
        ],
    )(x2d, agg2d, g.reshape(1, D), b.reshape(1, D), W1, b1.reshape(1, h4),
      W2, b2.reshape(1, D))


def _block_head_body(x_ref, agg_ref, g_ref, b_ref, w1_ref, b1_ref, w2_ref,
                     b2_ref, hg_ref, hb_ref, wh_ref, bh_ref, out_ref):
    h = _ln(agg_ref[...], g_ref[...], b_ref[...])
    u = jax.nn.gelu(jnp.dot(h, w1_ref[...], preferred_element_type=jnp.float32)
                    + b1_ref[...])
    y = jnp.dot(u, w2_ref[...], preferred_element_type=jnp.float32) + b2_ref[...]
    x = x_ref[...] + y
    h2 = _ln(x, hg_ref[...], hb_ref[...])
    out_ref[...] = (jnp.dot(h2, wh_ref[...], preferred_element_type=jnp.float32)
                    + bh_ref[...])


def _mlp_block_head(x2d, agg2d, g, b, W1, b1, W2, b2, hg, hb, W_head, b_head):
    r = 2048
    h4 = 4 * D
    return pl.pallas_call(
        _block_head_body,
        grid=(NROWS // r,),
        in_specs=[
            pl.BlockSpec((r, D), lambda i: (i, 0)),
            pl.BlockSpec((r, D), lambda i: (i, 0)),
            pl.BlockSpec((1, D), lambda i: (0, 0)),
            pl.BlockSpec((1, D), lambda i: (0, 0)),
            pl.BlockSpec((D, h4), lambda i: (0, 0)),
            pl.BlockSpec((1, h4), lambda i: (0, 0)),
            pl.BlockSpec((h4, D), lambda i: (0, 0)),
            pl.BlockSpec((1, D), lambda i: (0, 0)),
            pl.BlockSpec((1, D), lambda i: (0, 0)),
            pl.BlockSpec((1, D), lambda i: (0, 0)),
            pl.BlockSpec((D, OUT), lambda i: (0, 0)),
            pl.BlockSpec((1, OUT), lambda i: (0, 0)),
        ],
        out_specs=pl.BlockSpec((r, OUT), lambda i: (i, 0)),
        out_shape=jax.ShapeDtypeStruct((NROWS, OUT), jnp.float32),
    )(x2d, agg2d, g.reshape(1, D), b.reshape(1, D), W1, b1.reshape(1, h4),
      W2, b2.reshape(1, D), hg.reshape(1, D), hb.reshape(1, D), W_head,
      b_head.reshape(1, OUT))


def kernel(latent_token, adj, W_in, b_in, pos_embed, w_nb, ln1_g, ln1_b,
           W1, b1, W2, b2, lnh_g, lnh_b, W_head, b_head):
    depth = w_nb.shape[0]
    nat128 = jnp.asarray(_NAT128)
    nat512 = jnp.asarray(_NAT512)
    pos_pad = jnp.zeros((NPAD, D), jnp.float32).at[:N].set(pos_embed[0])
    adj_flat = jnp.zeros((NPAD, K), jnp.int32).at[:N].set(
        adj.astype(jnp.int32)).reshape(NW, N_CHUNKS, CK)
    def _as_i32(b16):
        return lax.bitcast_convert_type(
            b16.reshape(NPAD, ROWW // 2, 2), jnp.int32)

    x, xb = _init_x(latent_token, W_in, b_in, pos_pad)
    x = x.reshape(NPAD, ROWW)
    xb = _as_i32(xb.reshape(NPAD, ROWW))
    y = None
    for i in range(depth):
        w_st = jnp.tile(w_nb[i], (1, B)).astype(jnp.float32)[:, nat512]
        agg = _gather_agg(xb, adj_flat, w_st)
        g_st = ln1_g[i][nat128]
        b_st = ln1_b[i][nat128]
        W1_st = W1[i][nat128, :]
        if i < depth - 1:
            x2, xb2 = _mlp_block(x.reshape(NROWS, D), agg.reshape(NROWS, D),
                                 g_st, b_st, W1_st, b1[i], W2[i], b2[i])
            x = x2.reshape(NPAD, ROWW)
            xb = _as_i32(xb2.reshape(NPAD, ROWW))
        else:
            y = _mlp_block_head(x.reshape(NROWS, D), agg.reshape(NROWS, D),
                                g_st, b_st, W1_st, b1[i], W2[i], b2[i],
                                lnh_g, lnh_b, W_head, b_head)
    out = y.reshape(NPAD, B, OUT)[:N]
    return jnp.transpose(out, (1, 2, 0))


# speedup vs baseline: 2.9816x; 2.9816x over previous
import functools

import jax
import jax.numpy as jnp
from jax import lax
from jax.experimental import pallas as pl
from jax.experimental.pallas import tpu as pltpu
from jax.experimental.pallas import tpu_sc as plsc

N = 10000
K = 16
D = 128
B = 4
OUT = 2

NC, NS, L = 2, 16, 16
NW = NC * NS
ROWW = B * D
HROW = ROWW // 2
NPAD = 10240
PER_W = NPAD // NW
CHUNK = 8
N_CHUNKS = PER_W // CHUNK
NROWS = NPAD * B
CK = CHUNK * K
NH = N_CHUNKS // 2


def _gather_agg_body(x_hbm, adj_hbm, w_hbm, out_hbm, adj_v, rows_a, rows_b,
                     acc_a, acc_b, w_v, sem_a, sem_b, sem_oa, sem_ob):
    cid = lax.axis_index("c")
    sid = lax.axis_index("s")
    wid = sid * NC + cid
    base = wid * PER_W
    pltpu.sync_copy(w_hbm, w_v)
    pltpu.sync_copy(adj_hbm.at[wid], adj_v)

    def gather(c, buf, sem):
        return pltpu.async_copy(x_hbm.at[adj_v.at[c]], buf, sem)

    def wait_gather(buf, sem):
        pltpu.make_async_copy(x_hbm.at[pl.ds(0, CK)], buf, sem).wait()

    def wait_scatter(acc, sem):
        pltpu.make_async_copy(acc, out_hbm.at[pl.ds(0, CHUNK)], sem).wait()

    def compute(c, buf, acc):
        def vbody(v, carry):
            wv = [w_v[k, pl.ds(v * L, L)] for k in range(K)]
            for j in range(CHUNK):
                r0 = j * K
                for bb in range(B // 2):
                    xi = buf[r0, pl.ds(bb * D + v * L, L)]
                    te = plsc.bitcast(xi << 16, jnp.float32) * wv[0]
                    to = plsc.bitcast(xi & -65536, jnp.float32) * wv[0]
                    for k in range(1, K):
                        xi = buf[r0 + k, pl.ds(bb * D + v * L, L)]
                        te = te + plsc.bitcast(xi << 16, jnp.float32) * wv[k]
                        to = to + plsc.bitcast(xi & -65536, jnp.float32) * wv[k]
                    acc[j, pl.ds(bb * 2 * D + v * L, L)] = te
                    acc[j, pl.ds(bb * 2 * D + D + v * L, L)] = to
            return carry

        lax.fori_loop(0, D // L, vbody, 0)
        return pltpu.async_copy(
            acc, out_hbm.at[pl.ds(base + c * CHUNK, CHUNK)],
            sem_oa if acc is acc_a else sem_ob)

    gather(0, rows_a, sem_a)

    def body(t, carry):
        c0 = 2 * t
        c1 = 2 * t + 1
        gather(c1, rows_b, sem_b)
        wait_gather(rows_a, sem_a)

        @pl.when(t > 0)
        def _():
            wait_scatter(acc_a, sem_oa)
        compute(c0, rows_a, acc_a)

        @pl.when(t < NH - 1)
        def _():
            gather(c0 + 2, rows_a, sem_a)
        wait_gather(rows_b, sem_b)

        @pl.when(t > 0)
        def _():
            wait_scatter(acc_b, sem_ob)
        compute(c1, rows_b, acc_b)
        return carry

    lax.fori_loop(0, NH, body, 0)
    wait_scatter(acc_a, sem_oa)
    wait_scatter(acc_b, sem_ob)


@functools.partial(jax.jit, static_argnames=())
def _gather_agg(xb_rows, adj_w, w_feat):
    mesh = plsc.VectorSubcoreMesh(core_axis_name="c", subcore_axis_name="s")
    return pl.kernel(
        _gather_agg_body,
        out_type=jax.ShapeDtypeStruct((NPAD, ROWW), jnp.float32),
        mesh=mesh,
        compiler_params=pltpu.CompilerParams(needs_layout_passes=False),
        scratch_types=[
            pltpu.VMEM((N_CHUNKS, CK), jnp.int32),
            pltpu.VMEM((CK, HROW), jnp.int32),
            pltpu.VMEM((CK, HROW), jnp.int32),
            pltpu.VMEM((CHUNK, ROWW), jnp.float32),
            pltpu.VMEM((CHUNK, ROWW), jnp.float32),
            pltpu.VMEM((K, D), jnp.float32),
            pltpu.SemaphoreType.DMA,
            pltpu.SemaphoreType.DMA,
            pltpu.SemaphoreType.DMA,
            pltpu.SemaphoreType.DMA,
        ],
    )(xb_rows, adj_w, w_feat)


def _pack_pairs(x):
    u = lax.bitcast_convert_type(x, jnp.uint32)
    r2 = u.shape[0] // 2
    u = u.reshape(r2, 2, u.shape[1])
    one = jnp.uint32(1)
    half = jnp.uint32(0x7FFF)

    def rne(t):
        return (t + half + ((t >> 16) & one)) >> 16

    packed = (rne(u[:, 1, :]) << 16) | rne(u[:, 0, :])
    return lax.bitcast_convert_type(packed, jnp.int32)


def _init_body(lat_ref, win_ref, bin_ref, pos_ref, out_ref, outb_ref):
    x0 = jnp.dot(lat_ref[...], win_ref[...],
                 preferred_element_type=jnp.float32) + bin_ref[...]
    x = pos_ref[...][:, None, :] + x0[None, :, :]
    out_ref[...] = x
    tn = x.shape[0]
    outb_ref[...] = _pack_pairs(x.reshape(tn * B, D)).reshape(tn, B // 2, D)


def _init_x(latent, W_in, b_in, pos_pad):
    tn = 1024
    return pl.pallas_call(
        _init_body,
        grid=(NPAD // tn,),
        in_specs=[
            pl.BlockSpec((B, W_in.shape[0]), lambda i: (0, 0)),
            pl.BlockSpec((W_in.shape[0], D), lambda i: (0, 0)),
            pl.BlockSpec((1, D), lambda i: (0, 0)),
            pl.BlockSpec((tn, D), lambda i: (i, 0)),
        ],
        out_specs=[
            pl.BlockSpec((tn, B, D), lambda i: (i, 0, 0)),
            pl.BlockSpec((tn, B // 2, D), lambda i: (i, 0, 0)),
        ],
        out_shape=[
            jax.ShapeDtypeStruct((NPAD, B, D), jnp.float32),
            jax.ShapeDtypeStruct((NPAD, B // 2, D), jnp.int32),
        ],
    )(latent, W_in, b_in.reshape(1, D), pos_pad)


def _ln(x, g, b):
    m = jnp.mean(x, axis=-1, keepdims=True)
    v = jnp.mean((x - m) ** 2, axis=-1, keepdims=True)
    return (x - m) * lax.rsqrt(v + 1e-5) * g + b


def _block_body(x_ref, agg_ref, g_ref, b_ref, w1_ref, b1_ref, w2_ref, b2_ref,
                out_ref, outb_ref):
    h = _ln(agg_ref[...], g_ref[...], b_ref[...])
    u = jax.nn.gelu(jnp.dot(h, w1_ref[...], preferred_element_type=jnp.float32)
                    + b1_ref[...])
    y = jnp.dot(u, w2_ref[...], preferred_element_type=jnp.float32) + b2_ref[...]
    x = x_ref[...] + y
    out_ref[...] = x
    outb_ref[...] = _pack_pairs(x)


def _mlp_block(x2d, agg2d, g, b, W1, b1, W2, b2):
    r = 2048
    h4 = 4 * D
    return pl.pallas_call(
        _block_body,
        grid=(NROWS // r,),
        in_specs=[
            pl.BlockSpec((r, D), lambda i: (i, 0)),
            pl.BlockSpec((r, D), lambda i: (i, 0)),
            pl.BlockSpec((1, D), lambda i: (0, 0)),
            pl.BlockSpec((1, D), lambda i: (0, 0)),
            pl.BlockSpec((D, h4), lambda i: (0, 0)),
            pl.BlockSpec((1, h4), lambda i: (0, 0)),
            pl.BlockSpec((h4, D), lambda i: (0, 0)),
            pl.BlockSpec((1, D), lambda i: (0, 0)),
        ],
        out_specs=[
            pl.BlockSpec((r, D), lambda i: (i, 0)),
            pl.BlockSpec((r // 2, D), lambda i: (i, 0)),
        ],
        out_shape=[
            jax.ShapeDtypeStruct((NROWS, D), jnp.float32),
            jax.ShapeDtypeStruct((NROWS // 2, D), jnp.int32),
        ],
    )(x2d, agg2d, g.reshape(1, D), b.reshape(1, D), W1, b1.reshape(1, h4),
      W2, b2.reshape(1, D))


def _block_head_body(x_ref, agg_ref, g_ref, b_ref, w1_ref, b1_ref, w2_ref,
                     b2_ref, hg_ref, hb_ref, wh_ref, bh_ref, out_ref):
    h = _ln(agg_ref[...], g_ref[...], b_ref[...])
    u = jax.nn.gelu(jnp.dot(h, w1_ref[...], preferred_element_type=jnp.float32)
                    + b1_ref[...])
    y = jnp.dot(u, w2_ref[...], preferred_element_type=jnp.float32) + b2_ref[...]
    x = x_ref[...] + y
    h2 = _ln(x, hg_ref[...], hb_ref[...])
    out_ref[...] = (jnp.dot(h2, wh_ref[...], preferred_element_type=jnp.float32)
                    + bh_ref[...])


def _mlp_block_head(x2d, agg2d, g, b, W1, b1, W2, b2, hg, hb, W_head, b_head):
    r = 2048
    h4 = 4 * D
    return pl.pallas_call(
        _block_head_body,
        grid=(NROWS // r,),
        in_specs=[
            pl.BlockSpec((r, D), lambda i: (i, 0)),
            pl.BlockSpec((r, D), lambda i: (i, 0)),
            pl.BlockSpec((1, D), lambda i: (0, 0)),
            pl.BlockSpec((1, D), lambda i: (0, 0)),
            pl.BlockSpec((D, h4), lambda i: (0, 0)),
            pl.BlockSpec((1, h4), lambda i: (0, 0)),
            pl.BlockSpec((h4, D), lambda i: (0, 0)),
            pl.BlockSpec((1, D), lambda i: (0, 0)),
            pl.BlockSpec((1, D), lambda i: (0, 0)),
            pl.BlockSpec((1, D), lambda i: (0, 0)),
            pl.BlockSpec((D, OUT), lambda i: (0, 0)),
            pl.BlockSpec((1, OUT), lambda i: (0, 0)),
        ],
        out_specs=pl.BlockSpec((r, OUT), lambda i: (i, 0)),
        out_shape=jax.ShapeDtypeStruct((NROWS, OUT), jnp.float32),
    )(x2d, agg2d, g.reshape(1, D), b.reshape(1, D), W1, b1.reshape(1, h4),
      W2, b2.reshape(1, D), hg.reshape(1, D), hb.reshape(1, D), W_head,
      b_head.reshape(1, OUT))


def kernel(latent_token, adj, W_in, b_in, pos_embed, w_nb, ln1_g, ln1_b,
           W1, b1, W2, b2, lnh_g, lnh_b, W_head, b_head):
    depth = w_nb.shape[0]
    pos_pad = jnp.zeros((NPAD, D), jnp.float32).at[:N].set(pos_embed[0])
    adj_flat = jnp.zeros((NPAD, K), jnp.int32).at[:N].set(
        adj.astype(jnp.int32)).reshape(NW, N_CHUNKS, CK)
    x, xb = _init_x(latent_token, W_in, b_in, pos_pad)
    x = x.reshape(NPAD, ROWW)
    xb = xb.reshape(NPAD, HROW)
    y = None
    for i in range(depth):
        agg = _gather_agg(xb, adj_flat, w_nb[i].astype(jnp.float32))
        if i < depth - 1:
            x2, xb2 = _mlp_block(x.reshape(NROWS, D), agg.reshape(NROWS, D),
                                 ln1_g[i], ln1_b[i], W1[i], b1[i], W2[i], b2[i])
            x = x2.reshape(NPAD, ROWW)
            xb = xb2.reshape(NPAD, HROW)
        else:
            y = _mlp_block_head(x.reshape(NROWS, D), agg.reshape(NROWS, D),
                                ln1_g[i], ln1_b[i], W1[i], b1[i], W2[i], b2[i],
                                lnh_g, lnh_b, W_head, b_head)
    out = y.reshape(NPAD, B, OUT)[:N]
    return jnp.transpose(out, (1, 2, 0))
